# Initial kernel scaffold; baseline (speedup 1.0000x reference)
import jax
import jax.numpy as jnp
from jax.experimental import pallas as pl


def kernel(x, edge_index, W1, b1, W2, b2):
    raise NotImplementedError("write your pallas kernel here")


# speedup vs baseline: 6.7102x; 6.7102x over previous
import functools

import jax
import jax.numpy as jnp
from jax import lax
from jax.experimental import pallas as pl
from jax.experimental.pallas import tpu as pltpu
from jax.experimental.pallas import tpu_sc as plsc

_N = 100000
_E = 1600000
_D = 32

_NC = 2
_NS = 16
_HALF = _N // _NC
_DUMMY = _HALF
_PER_TEC = 3128
_ACC_ROWS = _PER_TEC * _NS

_CH = 512
_GJ = _CH // 128
_NCHUNK = 196
_Q = _CH * _NCHUNK
_EPAD = _Q * _NS
_ZF = _PER_TEC // _CH
_ZT = _PER_TEC % _CH


def _seg_body(h_hbm, src_hbm, dst_hbm, out_hbm,
              src1, dst1, idx2d, rows_v, acc, sem_g, sem_s):
    c = lax.axis_index("c")
    s = lax.axis_index("s")
    base = c * _HALF

    def _zb(t, carry):
        rows_v[t // 2, pl.ds((t % 2) * 16, 16)] = jnp.zeros((16,), jnp.float32)
        return carry
    lax.fori_loop(0, _CH * 2, _zb, 0)

    a0 = s * _PER_TEC
    def _zc(t, carry):
        pltpu.sync_copy(rows_v, acc.at[pl.ds(a0 + t * _CH, _CH)])
        return carry
    lax.fori_loop(0, _ZF, _zc, 0)
    pltpu.sync_copy(rows_v.at[pl.ds(0, _ZT)],
                    acc.at[pl.ds(a0 + _ZF * _CH, _ZT)])
    plsc.subcore_barrier()

    e0 = s * _Q
    def _chunk(t, carry):
        off = e0 + t * _CH
        pltpu.sync_copy(src_hbm.at[pl.ds(off, _CH)], src1)
        pltpu.sync_copy(dst_hbm.at[pl.ds(off, _CH)], dst1)
        gathers = [
            pltpu.async_copy(h_hbm.at[src1.at[pl.ds(j * 128, 128)]],
                             rows_v.at[pl.ds(j * 128, 128)], sem_g)
            for j in range(_GJ)
        ]
        def _vb(v, carry2):
            d = dst1[pl.ds(v * 16, 16)]
            local = d - base
            inb = (local >= 0) & (local < _HALF)
            idx2d[v // 8, pl.ds((v % 8) * 16, 16)] = jnp.where(inb, local, _DUMMY)
            return carry2
        lax.fori_loop(0, _CH // 16, _vb, 0)
        for g in gathers:
            g.wait()
        scatters = [
            pltpu.async_copy(rows_v.at[pl.ds(j * 128, 128)],
                             acc.at[idx2d.at[j]], sem_s, add=True)
            for j in range(_GJ)
        ]
        for sc in scatters:
            sc.wait()
        return carry
    lax.fori_loop(0, _NCHUNK, _chunk, 0)

    plsc.subcore_barrier()

    w0 = s * _PER_TEC
    def _wb(t, carry):
        pltpu.sync_copy(acc.at[pl.ds(w0 + t * _CH, _CH)], rows_v)
        pltpu.sync_copy(rows_v, out_hbm.at[pl.ds(base + w0 + t * _CH, _CH)])
        return carry
    lax.fori_loop(0, _ZF, _wb, 0)

    @pl.when(s < _NS - 1)
    def _full_tail():
        pltpu.sync_copy(acc.at[pl.ds(w0 + _ZF * _CH, _ZT)],
                        rows_v.at[pl.ds(0, _ZT)])
        pltpu.sync_copy(rows_v.at[pl.ds(0, _ZT)],
                        out_hbm.at[pl.ds(base + w0 + _ZF * _CH, _ZT)])

    @pl.when(s == _NS - 1)
    def _short_tail():
        _lt = _HALF - (_NS - 1) * _PER_TEC - _ZF * _CH
        _l0 = (_NS - 1) * _PER_TEC + _ZF * _CH
        pltpu.sync_copy(acc.at[pl.ds(_l0, _lt)], rows_v.at[pl.ds(0, _lt)])
        pltpu.sync_copy(rows_v.at[pl.ds(0, _lt)],
                        out_hbm.at[pl.ds(base + _l0, _lt)])


def _segment_sum(h, srcp, dstp):
    mesh = plsc.VectorSubcoreMesh(core_axis_name="c", subcore_axis_name="s")
    k = pl.kernel(
        _seg_body,
        out_type=jax.ShapeDtypeStruct((_N, _D), jnp.float32),
        mesh=mesh,
        scratch_types=[
            pltpu.VMEM((_CH,), jnp.int32),
            pltpu.VMEM((_CH,), jnp.int32),
            pltpu.VMEM((_GJ, 128), jnp.int32),
            pltpu.VMEM((_CH, _D), jnp.float32),
            pltpu.VMEM_SHARED((_ACC_ROWS, _D), jnp.float32),
            pltpu.SemaphoreType.DMA,
            pltpu.SemaphoreType.DMA,
        ],
        compiler_params=pltpu.CompilerParams(use_tc_tiling_on_sc=False),
    )
    return k(h, srcp, dstp)


def _lin_body(x_ref, w_ref, b_ref, o_ref, *, relu_in):
    xb = x_ref[...]
    if relu_in:
        xb = jnp.maximum(xb, 0.0)
    o_ref[...] = (
        jnp.dot(xb, w_ref[...], preferred_element_type=jnp.float32) + b_ref[...]
    )


def _linear(x, w, b, relu_in):
    blk = 2000
    grid = (_N // blk,)
    return pl.pallas_call(
        functools.partial(_lin_body, relu_in=relu_in),
        grid=grid,
        in_specs=[
            pl.BlockSpec((blk, _D), lambda i: (i, 0)),
            pl.BlockSpec((_D, _D), lambda i: (0, 0)),
            pl.BlockSpec((1, _D), lambda i: (0, 0)),
        ],
        out_specs=pl.BlockSpec((blk, _D), lambda i: (i, 0)),
        out_shape=jax.ShapeDtypeStruct((_N, _D), jnp.float32),
    )(x, w, b.reshape(1, _D))


def kernel(x, edge_index, W1, b1, W2, b2):
    pad = _EPAD - _E
    srcp = jnp.concatenate([edge_index[0], jnp.zeros((pad,), jnp.int32)])
    dstp = jnp.concatenate([edge_index[1], jnp.full((pad,), -1, jnp.int32)])
    h1 = _linear(x, W1, b1, relu_in=False)
    agg1 = _segment_sum(h1, srcp, dstp)
    h2 = _linear(agg1, W2, b2, relu_in=True)
    return _segment_sum(h2, srcp, dstp)
